# baseline (device time: 39771 ns/iter reference)
import jax
import jax.numpy as jnp
from jax import lax
from jax.experimental import pallas as pl
from jax.experimental.pallas import tpu as pltpu

N_DEV = 32
N_ROUNDS = 5
E_PER_DEV = 2


def kernel(x, router_W, route_idx, expert_W, shared_W):
    n, d = x.shape
    h = expert_W.shape[-1]

    def body(x_ref, rw_ref, idx_ref, ew_ref, sw_ref, out_ref,
             recv_buf, send_sems, recv_sems):
        my_i = lax.axis_index("i")

        barrier_sem = pltpu.get_barrier_semaphore()
        for k in range(N_ROUNDS):
            partner = my_i ^ (1 << k)
            pl.semaphore_signal(
                barrier_sem, inc=1,
                device_id=(partner,), device_id_type=pl.DeviceIdType.MESH,
            )
        pl.semaphore_wait(barrier_sem, N_ROUNDS)

        xv = x_ref[:, :]

        scores = jnp.dot(xv, rw_ref[:, :], preferred_element_type=jnp.float32)
        s_max = jnp.max(scores, axis=-1, keepdims=True)
        e = jnp.exp(scores - s_max)
        probs = e / jnp.sum(e, axis=-1, keepdims=True)
        idx = idx_ref[:, :]
        eids = lax.broadcasted_iota(jnp.int32, scores.shape, 1)
        p_sel = jnp.sum(jnp.where(eids == idx, probs, 0.0),
                        axis=-1, keepdims=True)

        y0 = jnp.dot(xv, ew_ref[0], preferred_element_type=jnp.float32)
        y1 = jnp.dot(xv, ew_ref[1], preferred_element_type=jnp.float32)
        w0 = jnp.where(idx == 2 * my_i, p_sel, 0.0)
        w1 = jnp.where(idx == 2 * my_i + 1, p_sel, 0.0)
        out_ref[:, :] = w0 * y0 + w1 * y1

        for k in range(N_ROUNDS):
            partner = my_i ^ (1 << k)
            rdma = pltpu.make_async_remote_copy(
                src_ref=out_ref,
                dst_ref=recv_buf.at[k],
                send_sem=send_sems.at[k],
                recv_sem=recv_sems.at[k],
                device_id=(partner,),
                device_id_type=pl.DeviceIdType.MESH,
            )
            rdma.start()
            rdma.wait()
            out_ref[:, :] = out_ref[:, :] + recv_buf[k]

        out_ref[:, :] = out_ref[:, :] + jnp.dot(
            xv, sw_ref[:, :], preferred_element_type=jnp.float32)

    return pl.pallas_call(
        body,
        out_shape=jax.ShapeDtypeStruct((n, h), jnp.float32),
        in_specs=[pl.BlockSpec(memory_space=pltpu.VMEM)] * 5,
        out_specs=pl.BlockSpec(memory_space=pltpu.VMEM),
        scratch_shapes=[
            pltpu.VMEM((N_ROUNDS, n, h), jnp.float32),
            pltpu.SemaphoreType.DMA((N_ROUNDS,)),
            pltpu.SemaphoreType.DMA((N_ROUNDS,)),
        ],
        compiler_params=pltpu.CompilerParams(collective_id=0),
    )(x, router_W, route_idx, expert_W, shared_W)


# device time: 29752 ns/iter; 1.3368x vs baseline; 1.3368x over previous
import jax
import jax.numpy as jnp
from jax import lax
from jax.experimental import pallas as pl
from jax.experimental.pallas import tpu as pltpu

N_DEV = 32
N_ROUNDS = 5
E_PER_DEV = 2


def kernel(x, router_W, route_idx, expert_W, shared_W):
    n, d = x.shape
    h = expert_W.shape[-1]

    def body(x_ref, rw_ref, idx_ref, ew_ref, sw_ref, out_ref,
             acc_ref, recv_buf, send_sems, recv_sems):
        my_i = lax.axis_index("i")

        barrier_sem = pltpu.get_barrier_semaphore()
        for k in range(N_ROUNDS):
            partner = my_i ^ (1 << k)
            pl.semaphore_signal(
                barrier_sem, inc=1,
                device_id=(partner,), device_id_type=pl.DeviceIdType.MESH,
            )
        pl.semaphore_wait(barrier_sem, N_ROUNDS)

        xv = x_ref[:, :]

        scores = jnp.dot(xv, rw_ref[:, :], preferred_element_type=jnp.float32)
        s_max = jnp.max(scores, axis=-1, keepdims=True)
        e = jnp.exp(scores - s_max)
        probs = e / jnp.sum(e, axis=-1, keepdims=True)
        idx = idx_ref[:, :]
        eids = lax.broadcasted_iota(jnp.int32, scores.shape, 1)
        p_sel = jnp.sum(jnp.where(eids == idx, probs, 0.0),
                        axis=-1, keepdims=True)

        y0 = jnp.dot(xv, ew_ref[0], preferred_element_type=jnp.float32)
        y1 = jnp.dot(xv, ew_ref[1], preferred_element_type=jnp.float32)
        w0 = jnp.where(idx == 2 * my_i, p_sel, 0.0)
        w1 = jnp.where(idx == 2 * my_i + 1, p_sel, 0.0)
        acc_ref[:, :] = (w0 * y0 + w1 * y1).astype(jnp.bfloat16)

        for k in range(N_ROUNDS):
            partner = my_i ^ (1 << k)
            rdma = pltpu.make_async_remote_copy(
                src_ref=acc_ref,
                dst_ref=recv_buf.at[k],
                send_sem=send_sems.at[k],
                recv_sem=recv_sems.at[k],
                device_id=(partner,),
                device_id_type=pl.DeviceIdType.MESH,
            )
            rdma.start()
            if k == 0:
                out_ref[:, :] = jnp.dot(
                    xv, sw_ref[:, :], preferred_element_type=jnp.float32)
            rdma.wait()
            acc_ref[:, :] = acc_ref[:, :] + recv_buf[k]

        out_ref[:, :] = out_ref[:, :] + acc_ref[:, :].astype(jnp.float32)

    return pl.pallas_call(
        body,
        out_shape=jax.ShapeDtypeStruct((n, h), jnp.float32),
        in_specs=[pl.BlockSpec(memory_space=pltpu.VMEM)] * 5,
        out_specs=pl.BlockSpec(memory_space=pltpu.VMEM),
        scratch_shapes=[
            pltpu.VMEM((n, h), jnp.bfloat16),
            pltpu.VMEM((N_ROUNDS, n, h), jnp.bfloat16),
            pltpu.SemaphoreType.DMA((N_ROUNDS,)),
            pltpu.SemaphoreType.DMA((N_ROUNDS,)),
        ],
        compiler_params=pltpu.CompilerParams(collective_id=0),
    )(x, router_W, route_idx, expert_W, shared_W)


# device time: 25942 ns/iter; 1.5331x vs baseline; 1.1469x over previous
import jax
import jax.numpy as jnp
from jax import lax
from jax.experimental import pallas as pl
from jax.experimental.pallas import tpu as pltpu

N_DEV = 32
N_ROUNDS = 5
E_PER_DEV = 2


def kernel(x, router_W, route_idx, expert_W, shared_W):
    n, d = x.shape
    h = expert_W.shape[-1]

    half = n // 2

    def body(x_ref, rw_ref, idx_ref, ew_ref, sw_ref, out_ref,
             acc_ref, recv_a, recv_b,
             send_sems_a, recv_sems_a, send_sems_b, recv_sems_b):
        my_i = lax.axis_index("i")

        barrier_sem = pltpu.get_barrier_semaphore()
        for k in range(N_ROUNDS):
            partner = my_i ^ (1 << k)
            pl.semaphore_signal(
                barrier_sem, inc=1,
                device_id=(partner,), device_id_type=pl.DeviceIdType.MESH,
            )
        pl.semaphore_wait(barrier_sem, N_ROUNDS)

        xv = x_ref[:, :]

        scores = jnp.dot(xv, rw_ref[:, :], preferred_element_type=jnp.float32)
        s_max = jnp.max(scores, axis=-1, keepdims=True)
        e = jnp.exp(scores - s_max)
        probs = e / jnp.sum(e, axis=-1, keepdims=True)
        idx = idx_ref[:, :]
        eids = lax.broadcasted_iota(jnp.int32, scores.shape, 1)
        p_sel = jnp.sum(jnp.where(eids == idx, probs, 0.0),
                        axis=-1, keepdims=True)

        y0 = jnp.dot(xv, ew_ref[0], preferred_element_type=jnp.float32)
        y1 = jnp.dot(xv, ew_ref[1], preferred_element_type=jnp.float32)
        w0 = jnp.where(idx == 2 * my_i, p_sel, 0.0)
        w1 = jnp.where(idx == 2 * my_i + 1, p_sel, 0.0)
        acc_ref[:, :] = (w0 * y0 + w1 * y1).astype(jnp.bfloat16)

        for t in range(N_ROUNDS):
            pa = my_i ^ (1 << t)
            pb = my_i ^ (1 << ((t + 1) % N_ROUNDS))
            rdma_a = pltpu.make_async_remote_copy(
                src_ref=acc_ref.at[pl.ds(0, half)],
                dst_ref=recv_a.at[t],
                send_sem=send_sems_a.at[t],
                recv_sem=recv_sems_a.at[t],
                device_id=(pa,),
                device_id_type=pl.DeviceIdType.MESH,
            )
            rdma_b = pltpu.make_async_remote_copy(
                src_ref=acc_ref.at[pl.ds(half, half)],
                dst_ref=recv_b.at[t],
                send_sem=send_sems_b.at[t],
                recv_sem=recv_sems_b.at[t],
                device_id=(pb,),
                device_id_type=pl.DeviceIdType.MESH,
            )
            rdma_a.start()
            rdma_b.start()
            if t == 0:
                out_ref[:, :] = jnp.dot(
                    xv, sw_ref[:, :], preferred_element_type=jnp.float32)
            rdma_a.wait()
            acc_ref[pl.ds(0, half), :] = (
                acc_ref[pl.ds(0, half), :] + recv_a[t])
            rdma_b.wait()
            acc_ref[pl.ds(half, half), :] = (
                acc_ref[pl.ds(half, half), :] + recv_b[t])

        out_ref[:, :] = out_ref[:, :] + acc_ref[:, :].astype(jnp.float32)

    return pl.pallas_call(
        body,
        out_shape=jax.ShapeDtypeStruct((n, h), jnp.float32),
        in_specs=[pl.BlockSpec(memory_space=pltpu.VMEM)] * 5,
        out_specs=pl.BlockSpec(memory_space=pltpu.VMEM),
        scratch_shapes=[
            pltpu.VMEM((n, h), jnp.bfloat16),
            pltpu.VMEM((N_ROUNDS, n // 2, h), jnp.bfloat16),
            pltpu.VMEM((N_ROUNDS, n // 2, h), jnp.bfloat16),
            pltpu.SemaphoreType.DMA((N_ROUNDS,)),
            pltpu.SemaphoreType.DMA((N_ROUNDS,)),
            pltpu.SemaphoreType.DMA((N_ROUNDS,)),
            pltpu.SemaphoreType.DMA((N_ROUNDS,)),
        ],
        compiler_params=pltpu.CompilerParams(collective_id=0),
    )(x, router_W, route_idx, expert_W, shared_W)


# device time: 24233 ns/iter; 1.6412x vs baseline; 1.0705x over previous
import jax
import jax.numpy as jnp
from jax import lax
from jax.experimental import pallas as pl
from jax.experimental.pallas import tpu as pltpu

N_DEV = 32
N_ROUNDS = 5
E_PER_DEV = 2


def kernel(x, router_W, route_idx, expert_W, shared_W):
    n, d = x.shape
    h = expert_W.shape[-1]

    half = n // 2

    def body(x_ref, rw_ref, idx_ref, ew_ref, sw_ref, out_ref,
             acc_ref, recv_a, recv_b,
             send_sems_a, recv_sems_a, send_sems_b, recv_sems_b):
        my_i = lax.axis_index("i")

        def partner_for_round(r, p):
            if r == 0:
                return p ^ 1
            if r == 2:
                return p ^ 4
            if r == 3:
                return p ^ 8
            if r == 4:
                return p ^ 16
            z = p >> 3
            s = p & 7
            y = s >> 1
            x = (s & 1) ^ (y & 1)
            ny = y ^ 1
            ns = (ny << 1) | (x ^ (ny & 1))
            return (z << 3) | ns

        barrier_sem = pltpu.get_barrier_semaphore()
        for r in range(N_ROUNDS):
            pl.semaphore_signal(
                barrier_sem, inc=1,
                device_id=(partner_for_round(r, my_i),),
                device_id_type=pl.DeviceIdType.MESH,
            )
        pl.semaphore_wait(barrier_sem, N_ROUNDS)

        xv = x_ref[:, :]

        scores = jnp.dot(xv, rw_ref[:, :], preferred_element_type=jnp.float32)
        s_max = jnp.max(scores, axis=-1, keepdims=True)
        e = jnp.exp(scores - s_max)
        probs = e / jnp.sum(e, axis=-1, keepdims=True)
        idx = idx_ref[:, :]
        eids = lax.broadcasted_iota(jnp.int32, scores.shape, 1)
        p_sel = jnp.sum(jnp.where(eids == idx, probs, 0.0),
                        axis=-1, keepdims=True)

        y0 = jnp.dot(xv, ew_ref[0], preferred_element_type=jnp.float32)
        y1 = jnp.dot(xv, ew_ref[1], preferred_element_type=jnp.float32)
        w0 = jnp.where(idx == 2 * my_i, p_sel, 0.0)
        w1 = jnp.where(idx == 2 * my_i + 1, p_sel, 0.0)
        acc_ref[:, :] = (w0 * y0 + w1 * y1).astype(jnp.bfloat16)

        for t in range(N_ROUNDS):
            pa = partner_for_round(t, my_i)
            pb = partner_for_round((t + 3) % N_ROUNDS, my_i)
            rdma_a = pltpu.make_async_remote_copy(
                src_ref=acc_ref.at[pl.ds(0, half)],
                dst_ref=recv_a.at[t],
                send_sem=send_sems_a.at[t],
                recv_sem=recv_sems_a.at[t],
                device_id=(pa,),
                device_id_type=pl.DeviceIdType.MESH,
            )
            rdma_b = pltpu.make_async_remote_copy(
                src_ref=acc_ref.at[pl.ds(half, half)],
                dst_ref=recv_b.at[t],
                send_sem=send_sems_b.at[t],
                recv_sem=recv_sems_b.at[t],
                device_id=(pb,),
                device_id_type=pl.DeviceIdType.MESH,
            )
            rdma_a.start()
            rdma_b.start()
            if t == 0:
                out_ref[:, :] = jnp.dot(
                    xv, sw_ref[:, :], preferred_element_type=jnp.float32)
            rdma_a.wait()
            acc_ref[pl.ds(0, half), :] = (
                acc_ref[pl.ds(0, half), :] + recv_a[t])
            rdma_b.wait()
            acc_ref[pl.ds(half, half), :] = (
                acc_ref[pl.ds(half, half), :] + recv_b[t])

        out_ref[:, :] = out_ref[:, :] + acc_ref[:, :].astype(jnp.float32)

    return pl.pallas_call(
        body,
        out_shape=jax.ShapeDtypeStruct((n, h), jnp.float32),
        in_specs=[pl.BlockSpec(memory_space=pltpu.VMEM)] * 5,
        out_specs=pl.BlockSpec(memory_space=pltpu.VMEM),
        scratch_shapes=[
            pltpu.VMEM((n, h), jnp.bfloat16),
            pltpu.VMEM((N_ROUNDS, n // 2, h), jnp.bfloat16),
            pltpu.VMEM((N_ROUNDS, n // 2, h), jnp.bfloat16),
            pltpu.SemaphoreType.DMA((N_ROUNDS,)),
            pltpu.SemaphoreType.DMA((N_ROUNDS,)),
            pltpu.SemaphoreType.DMA((N_ROUNDS,)),
            pltpu.SemaphoreType.DMA((N_ROUNDS,)),
        ],
        compiler_params=pltpu.CompilerParams(collective_id=0),
    )(x, router_W, route_idx, expert_W, shared_W)


# device time: 23907 ns/iter; 1.6636x vs baseline; 1.0136x over previous
import jax
import jax.numpy as jnp
from jax import lax
from jax.experimental import pallas as pl
from jax.experimental.pallas import tpu as pltpu

N_DEV = 32
N_ROUNDS = 5
E_PER_DEV = 2


def kernel(x, router_W, route_idx, expert_W, shared_W):
    n, d = x.shape
    h = expert_W.shape[-1]

    half = n // 2

    def body(x_ref, rw_ref, idx_ref, ew_ref, sw_ref, out_ref,
             acc_ref, recv_a, recv_b,
             send_sems_a, recv_sems_a, send_sems_b, recv_sems_b):
        my_i = lax.axis_index("i")

        def partner_for_round(r, p):
            if r == 0:
                return p ^ 1
            if r == 2:
                return p ^ 4
            if r == 3:
                return p ^ 8
            if r == 4:
                return p ^ 16
            z = p >> 3
            s = p & 7
            y = s >> 1
            x = (s & 1) ^ (y & 1)
            ny = y ^ 1
            ns = (ny << 1) | (x ^ (ny & 1))
            return (z << 3) | ns

        barrier_sem = pltpu.get_barrier_semaphore()
        for r in range(N_ROUNDS):
            pl.semaphore_signal(
                barrier_sem, inc=1,
                device_id=(partner_for_round(r, my_i),),
                device_id_type=pl.DeviceIdType.MESH,
            )

        xv = x_ref[:, :]

        scores = jnp.dot(xv, rw_ref[:, :], preferred_element_type=jnp.float32)
        s_max = jnp.max(scores, axis=-1, keepdims=True)
        e = jnp.exp(scores - s_max)
        probs = e / jnp.sum(e, axis=-1, keepdims=True)
        idx = idx_ref[:, :]
        eids = lax.broadcasted_iota(jnp.int32, scores.shape, 1)
        p_sel = jnp.sum(jnp.where(eids == idx, probs, 0.0),
                        axis=-1, keepdims=True)

        w0 = jnp.where(idx == 2 * my_i, p_sel, 0.0)
        w1 = jnp.where(idx == 2 * my_i + 1, p_sel, 0.0)

        def partial_half(lo):
            xh = xv[lo:lo + half, :]
            y0 = jnp.dot(xh, ew_ref[0], preferred_element_type=jnp.float32)
            y1 = jnp.dot(xh, ew_ref[1], preferred_element_type=jnp.float32)
            return (w0[lo:lo + half] * y0
                    + w1[lo:lo + half] * y1).astype(jnp.bfloat16)

        for t in range(N_ROUNDS):
            pa = partner_for_round(t, my_i)
            pb = partner_for_round((t + 3) % N_ROUNDS, my_i)
            rdma_a = pltpu.make_async_remote_copy(
                src_ref=acc_ref.at[pl.ds(0, half)],
                dst_ref=recv_a.at[t],
                send_sem=send_sems_a.at[t],
                recv_sem=recv_sems_a.at[t],
                device_id=(pa,),
                device_id_type=pl.DeviceIdType.MESH,
            )
            rdma_b = pltpu.make_async_remote_copy(
                src_ref=acc_ref.at[pl.ds(half, half)],
                dst_ref=recv_b.at[t],
                send_sem=send_sems_b.at[t],
                recv_sem=recv_sems_b.at[t],
                device_id=(pb,),
                device_id_type=pl.DeviceIdType.MESH,
            )
            if t == 0:
                acc_ref[pl.ds(0, half), :] = partial_half(0)
                pl.semaphore_wait(barrier_sem, N_ROUNDS)
                rdma_a.start()
                acc_ref[pl.ds(half, half), :] = partial_half(half)
                rdma_b.start()
                out_ref[:, :] = jnp.dot(
                    xv, sw_ref[:, :], preferred_element_type=jnp.float32)
            else:
                rdma_a.start()
                rdma_b.start()
            rdma_a.wait()
            acc_ref[pl.ds(0, half), :] = (
                acc_ref[pl.ds(0, half), :] + recv_a[t])
            rdma_b.wait()
            acc_ref[pl.ds(half, half), :] = (
                acc_ref[pl.ds(half, half), :] + recv_b[t])

        out_ref[:, :] = out_ref[:, :] + acc_ref[:, :].astype(jnp.float32)

    return pl.pallas_call(
        body,
        out_shape=jax.ShapeDtypeStruct((n, h), jnp.float32),
        in_specs=[pl.BlockSpec(memory_space=pltpu.VMEM)] * 5,
        out_specs=pl.BlockSpec(memory_space=pltpu.VMEM),
        scratch_shapes=[
            pltpu.VMEM((n, h), jnp.bfloat16),
            pltpu.VMEM((N_ROUNDS, n // 2, h), jnp.bfloat16),
            pltpu.VMEM((N_ROUNDS, n // 2, h), jnp.bfloat16),
            pltpu.SemaphoreType.DMA((N_ROUNDS,)),
            pltpu.SemaphoreType.DMA((N_ROUNDS,)),
            pltpu.SemaphoreType.DMA((N_ROUNDS,)),
            pltpu.SemaphoreType.DMA((N_ROUNDS,)),
        ],
        compiler_params=pltpu.CompilerParams(collective_id=0),
    )(x, router_W, route_idx, expert_W, shared_W)
